# baseline (speedup 1.0000x reference)
import functools

import jax
import jax.numpy as jnp
from jax import lax
from jax.experimental import pallas as pl
from jax.experimental.pallas import tpu as pltpu
from jax.experimental.pallas import tpu_sc as plsc

_N_CODES = 8192
_D = 32
_BR = 512
_BC = 1024


def _tc_argmin_body(zn_ref, cb_ref, a_ref, b_ref, idx_ref):
    zn = zn_ref[...]
    a = a_ref[...]

    def chunk(j, carry):
        m, bi = carry
        cbj = cb_ref[pl.ds(j * _BC, _BC), :]
        dot = lax.dot_general(
            zn, cbj, (((1,), (1,)), ((), ())),
            preferred_element_type=jnp.float32)
        d = (a + b_ref[:, pl.ds(j * _BC, _BC)]) - 2.0 * dot
        cm = jnp.min(d, axis=1, keepdims=True)
        io = lax.broadcasted_iota(jnp.int32, (_BR, _BC), 1)
        ci = jnp.min(jnp.where(d == cm, io, _BC), axis=1, keepdims=True)
        ci = ci + j * _BC
        better = cm < m
        return jnp.where(better, cm, m), jnp.where(better, ci, bi)

    m0 = jnp.full((_BR, 1), jnp.inf, dtype=jnp.float32)
    i0 = jnp.zeros((_BR, 1), dtype=jnp.int32)
    _, bi = lax.fori_loop(0, _N_CODES // _BC, chunk, (m0, i0))
    idx_ref[...] = bi[:, 0]


def _tc_argmin(zn, codebook, a, b):
    n_rows = zn.shape[0]
    grid = (n_rows // _BR,)
    return pl.pallas_call(
        _tc_argmin_body,
        grid=grid,
        in_specs=[
            pl.BlockSpec((_BR, _D), lambda i: (i, 0)),
            pl.BlockSpec((_N_CODES, _D), lambda i: (0, 0)),
            pl.BlockSpec((_BR, 1), lambda i: (i, 0)),
            pl.BlockSpec((1, _N_CODES), lambda i: (0, 0)),
        ],
        out_specs=pl.BlockSpec((_BR,), lambda i: (i,)),
        out_shape=jax.ShapeDtypeStruct((n_rows,), jnp.int32),
        compiler_params=pltpu.CompilerParams(
            dimension_semantics=("parallel",)),
    )(zn, codebook, a, b)


def _sc_gather(table, idx):
    info = plsc.get_sparse_core_info()
    nw = info.num_cores * info.num_subcores
    b = idx.shape[0]
    b_per_w = b // nw
    nc = info.num_cores
    mesh = plsc.VectorSubcoreMesh(core_axis_name="c", subcore_axis_name="s")

    @functools.partial(
        pl.kernel, mesh=mesh,
        out_type=jax.ShapeDtypeStruct((b, _D), jnp.float32),
        scratch_types=[
            pltpu.VMEM((b_per_w,), jnp.int32),
            pltpu.VMEM((b_per_w, _D), jnp.float32),
            pltpu.SemaphoreType.DMA,
        ],
        compiler_params=pltpu.CompilerParams(use_tc_tiling_on_sc=False),
    )
    def gather(table_hbm, idx_hbm, out_hbm, idx_v, rows_v, sem):
        wid = lax.axis_index("s") * nc + lax.axis_index("c")
        base = wid * b_per_w
        pltpu.sync_copy(idx_hbm.at[pl.ds(base, b_per_w)], idx_v)
        pltpu.async_copy(table_hbm.at[idx_v], rows_v, sem).wait()
        pltpu.sync_copy(rows_v, out_hbm.at[pl.ds(base, b_per_w)])

    return gather(table, idx)


def kernel(z, codebook):
    zn = z / jnp.clip(
        jnp.linalg.norm(z, ord=2, axis=-1, keepdims=True), 1e-12)
    z_flat = zn.reshape(-1, _D)
    a = jnp.sum(z_flat ** 2, axis=1, keepdims=True)
    b = jnp.sum(codebook ** 2, axis=1)[None, :]
    idx = _tc_argmin(z_flat, codebook, a, b)
    z_q = _sc_gather(codebook, idx)
    return (z_q, idx)
